# baseline (speedup 1.0000x reference)
import jax
import jax.numpy as jnp
from jax.experimental import pallas as pl
from jax.experimental.pallas import tpu as pltpu

_CHUNKS = 2


def _pos_kernel(row_ref, col_ref, out4d_ref, scratch, sems):
    b = out4d_ref.shape[0]
    f = col_ref.shape[1]
    h = row_ref.shape[0]
    w = col_ref.shape[0]
    m = f * h

    out_ref = out4d_ref.reshape(b, 2 * m, w)

    nb = 4
    br = m // nb
    bi0 = jax.lax.broadcasted_iota(jnp.int32, (br, f), 0)
    bi1 = jax.lax.broadcasted_iota(jnp.int32, (br, f), 1)

    for blk in range(nb):
        c0 = blk * (br // h)
        sel = (bi1 == c0 + bi0 // h).astype(jnp.float32)
        scratch[blk * br : (blk + 1) * br, :] = jax.lax.dot_general(
            sel, col_ref[...], (((1,), (1,)), ((), ())),
            precision=jax.lax.Precision.HIGHEST,
        )
    rows = m // _CHUNKS
    for i in range(b):
        for j in range(_CHUNKS):
            pltpu.make_async_copy(
                scratch.at[pl.ds(j * rows, rows)],
                out_ref.at[i, pl.ds(j * rows, rows)],
                sems.at[i, j],
            ).start()

    l0 = jax.lax.broadcasted_iota(jnp.int32, (br, w), 0)
    l1 = jax.lax.broadcasted_iota(jnp.int32, (br, w), 1)
    pick = l1 == l0 % h
    ones = jnp.ones((w, w), jnp.float32)
    for blk in range(nb):
        c0 = blk * (br // h)
        sel = (bi1 == c0 + bi0 // h).astype(jnp.float32)
        y_sel = jax.lax.dot_general(
            sel, row_ref[...], (((1,), (1,)), ((), ())),
            precision=jax.lax.Precision.HIGHEST,
        )
        y_pick = jnp.where(pick, y_sel, 0.0)
        scratch[m + blk * br : m + (blk + 1) * br, :] = jax.lax.dot(
            y_pick, ones, precision=jax.lax.Precision.HIGHEST
        )
    for i in range(b):
        for j in range(_CHUNKS):
            pltpu.make_async_copy(
                scratch.at[pl.ds(m + j * rows, rows)],
                out_ref.at[i, pl.ds(m + j * rows, rows)],
                sems.at[i, _CHUNKS + j],
            ).start()

    for i in range(b):
        for j in range(_CHUNKS):
            pltpu.make_async_copy(
                scratch.at[pl.ds(j * rows, rows)],
                out_ref.at[i, pl.ds(j * rows, rows)],
                sems.at[i, j],
            ).wait()
            pltpu.make_async_copy(
                scratch.at[pl.ds(m + j * rows, rows)],
                out_ref.at[i, pl.ds(m + j * rows, rows)],
                sems.at[i, _CHUNKS + j],
            ).wait()


def kernel(mask, row_embed, col_embed):
    b = mask.shape[0]
    h, w = mask.shape[-2], mask.shape[-1]
    f = col_embed.shape[-1]

    out = pl.pallas_call(
        _pos_kernel,
        in_specs=[
            pl.BlockSpec(memory_space=pltpu.MemorySpace.VMEM),
            pl.BlockSpec(memory_space=pltpu.MemorySpace.VMEM),
        ],
        out_specs=pl.BlockSpec(memory_space=pltpu.MemorySpace.HBM),
        out_shape=jax.ShapeDtypeStruct((b, 2 * f, h, w), jnp.float32),
        scratch_shapes=[
            pltpu.VMEM((2 * f * h, w), jnp.float32),
            pltpu.SemaphoreType.DMA((b, 2 * _CHUNKS)),
        ],
    )(row_embed, col_embed)
    return out


# speedup vs baseline: 1.2158x; 1.2158x over previous
import jax
import jax.numpy as jnp
from jax.experimental import pallas as pl
from jax.experimental.pallas import tpu as pltpu

_CHUNKS = 2


def _pos_kernel(row_ref, col_ref, out4d_ref, scratch, sems):
    b = out4d_ref.shape[0]
    f = col_ref.shape[1]
    h = row_ref.shape[0]
    w = col_ref.shape[0]
    m = f * h

    out_ref = out4d_ref.reshape(b, 2 * m, w)

    nb = 4
    br = m // nb
    bi0 = jax.lax.broadcasted_iota(jnp.int32, (br, f), 0)
    bi1 = jax.lax.broadcasted_iota(jnp.int32, (br, f), 1)

    for blk in range(nb):
        c0 = blk * (br // h)
        sel = (bi1 == c0 + bi0 // h).astype(jnp.float32)
        scratch[blk * br : (blk + 1) * br, :] = jax.lax.dot_general(
            sel, col_ref[...], (((1,), (1,)), ((), ())),
            precision=jax.lax.Precision.DEFAULT,
        )
    rows = m // _CHUNKS
    for i in range(b):
        for j in range(_CHUNKS):
            pltpu.make_async_copy(
                scratch.at[pl.ds(j * rows, rows)],
                out_ref.at[i, pl.ds(j * rows, rows)],
                sems.at[i, j],
            ).start()

    l0 = jax.lax.broadcasted_iota(jnp.int32, (br, w), 0)
    l1 = jax.lax.broadcasted_iota(jnp.int32, (br, w), 1)
    pick = l1 == l0 % h
    ones = jnp.ones((w, w), jnp.float32)
    for blk in range(nb):
        c0 = blk * (br // h)
        sel = (bi1 == c0 + bi0 // h).astype(jnp.float32)
        y_sel = jax.lax.dot_general(
            sel, row_ref[...], (((1,), (1,)), ((), ())),
            precision=jax.lax.Precision.DEFAULT,
        )
        y_pick = jnp.where(pick, y_sel, 0.0)
        scratch[m + blk * br : m + (blk + 1) * br, :] = jax.lax.dot(
            y_pick, ones, precision=jax.lax.Precision.DEFAULT
        )
    for i in range(b):
        for j in range(_CHUNKS):
            pltpu.make_async_copy(
                scratch.at[pl.ds(m + j * rows, rows)],
                out_ref.at[i, pl.ds(m + j * rows, rows)],
                sems.at[i, _CHUNKS + j],
            ).start()

    for i in range(b):
        for j in range(_CHUNKS):
            pltpu.make_async_copy(
                scratch.at[pl.ds(j * rows, rows)],
                out_ref.at[i, pl.ds(j * rows, rows)],
                sems.at[i, j],
            ).wait()
            pltpu.make_async_copy(
                scratch.at[pl.ds(m + j * rows, rows)],
                out_ref.at[i, pl.ds(m + j * rows, rows)],
                sems.at[i, _CHUNKS + j],
            ).wait()


def kernel(mask, row_embed, col_embed):
    b = mask.shape[0]
    h, w = mask.shape[-2], mask.shape[-1]
    f = col_embed.shape[-1]

    out = pl.pallas_call(
        _pos_kernel,
        in_specs=[
            pl.BlockSpec(memory_space=pltpu.MemorySpace.VMEM),
            pl.BlockSpec(memory_space=pltpu.MemorySpace.VMEM),
        ],
        out_specs=pl.BlockSpec(memory_space=pltpu.MemorySpace.HBM),
        out_shape=jax.ShapeDtypeStruct((b, 2 * f, h, w), jnp.float32),
        scratch_shapes=[
            pltpu.VMEM((2 * f * h, w), jnp.float32),
            pltpu.SemaphoreType.DMA((b, 2 * _CHUNKS)),
        ],
    )(row_embed, col_embed)
    return out


# speedup vs baseline: 6.8191x; 5.6087x over previous
import jax
import jax.numpy as jnp
from jax.experimental import pallas as pl
from jax.experimental.pallas import tpu as pltpu

_CHUNKS = 4


def _pos_kernel(row_ref, col_ref, out4d_ref, scratch, sems):
    b = out4d_ref.shape[0]
    f = col_ref.shape[1]
    h = row_ref.shape[0]
    w = col_ref.shape[0]
    n = h * w

    out_ref = out4d_ref.reshape(b, n, 2 * f)

    i0 = jax.lax.broadcasted_iota(jnp.int32, (n, w), 0)
    i1 = jax.lax.broadcasted_iota(jnp.int32, (n, w), 1)

    sel_x = (i1 == i0 % w).astype(jnp.float32)
    scratch[:, 0:f] = jax.lax.dot(sel_x, col_ref[...])
    sel_y = (i1 == i0 // w).astype(jnp.float32)
    scratch[:, f : 2 * f] = jax.lax.dot(sel_y, row_ref[...])

    rows = n // _CHUNKS
    for i in range(b):
        for j in range(_CHUNKS):
            pltpu.make_async_copy(
                scratch.at[pl.ds(j * rows, rows)],
                out_ref.at[i, pl.ds(j * rows, rows)],
                sems.at[i, j],
            ).start()
    for i in range(b):
        for j in range(_CHUNKS):
            pltpu.make_async_copy(
                scratch.at[pl.ds(j * rows, rows)],
                out_ref.at[i, pl.ds(j * rows, rows)],
                sems.at[i, j],
            ).wait()


def kernel(mask, row_embed, col_embed):
    b = mask.shape[0]
    h, w = mask.shape[-2], mask.shape[-1]
    f = col_embed.shape[-1]

    out = pl.pallas_call(
        _pos_kernel,
        in_specs=[
            pl.BlockSpec(memory_space=pltpu.MemorySpace.VMEM),
            pl.BlockSpec(memory_space=pltpu.MemorySpace.VMEM),
        ],
        out_specs=pl.BlockSpec(memory_space=pltpu.MemorySpace.HBM),
        out_shape=jax.ShapeDtypeStruct((b, h, w, 2 * f), jnp.float32),
        scratch_shapes=[
            pltpu.VMEM((h * w, 2 * f), jnp.float32),
            pltpu.SemaphoreType.DMA((b, _CHUNKS)),
        ],
    )(row_embed, col_embed)
    return jnp.transpose(out, (0, 3, 1, 2))


# speedup vs baseline: 7.0941x; 1.0403x over previous
import jax
import jax.numpy as jnp
from jax.experimental import pallas as pl
from jax.experimental.pallas import tpu as pltpu

_CHUNKS = 4


def _pos_kernel(row_ref, col_ref, out4d_ref, scratch, sems):
    b = out4d_ref.shape[0]
    f = col_ref.shape[1]
    h = row_ref.shape[0]
    w = col_ref.shape[0]
    n = h * w

    out_ref = out4d_ref.reshape(b, n, 2 * f)

    rows = n // _CHUNKS
    i0 = jax.lax.broadcasted_iota(jnp.int32, (rows, w), 0)
    i1 = jax.lax.broadcasted_iota(jnp.int32, (rows, w), 1)
    sel_x = (i1 == i0 % w).astype(jnp.float32)
    for j in range(_CHUNKS):
        sel_y = (i1 == i0 // w + j * (rows // w)).astype(jnp.float32)
        scratch[j * rows : (j + 1) * rows, 0:f] = jax.lax.dot(sel_x, col_ref[...])
        scratch[j * rows : (j + 1) * rows, f : 2 * f] = jax.lax.dot(
            sel_y, row_ref[...]
        )
        for i in range(b):
            pltpu.make_async_copy(
                scratch.at[pl.ds(j * rows, rows)],
                out_ref.at[i, pl.ds(j * rows, rows)],
                sems.at[i, j],
            ).start()
    for i in range(b):
        for j in range(_CHUNKS):
            pltpu.make_async_copy(
                scratch.at[pl.ds(j * rows, rows)],
                out_ref.at[i, pl.ds(j * rows, rows)],
                sems.at[i, j],
            ).wait()


def kernel(mask, row_embed, col_embed):
    b = mask.shape[0]
    h, w = mask.shape[-2], mask.shape[-1]
    f = col_embed.shape[-1]

    out = pl.pallas_call(
        _pos_kernel,
        in_specs=[
            pl.BlockSpec(memory_space=pltpu.MemorySpace.VMEM),
            pl.BlockSpec(memory_space=pltpu.MemorySpace.VMEM),
        ],
        out_specs=pl.BlockSpec(memory_space=pltpu.MemorySpace.HBM),
        out_shape=jax.ShapeDtypeStruct((b, h, w, 2 * f), jnp.float32),
        scratch_shapes=[
            pltpu.VMEM((h * w, 2 * f), jnp.float32),
            pltpu.SemaphoreType.DMA((b, _CHUNKS)),
        ],
    )(row_embed, col_embed)
    return jnp.transpose(out, (0, 3, 1, 2))
